# baseline (speedup 1.0000x reference)
import functools

import jax
import jax.numpy as jnp
from jax import lax
from jax.experimental import pallas as pl
from jax.experimental.pallas import tpu as pltpu
from jax.experimental.pallas import tpu_sc as plsc

VOCAB = 100000
IMG = 2048
HID = 128
TOTAL = VOCAB + IMG
PADDED = 102400

CH = 12800
GRID1 = 8

NTILES = 32
PER_TILE = PADDED // NTILES
CHUNK = 128
NCHUNK = PER_TILE // CHUNK


def _embsum_body(emb_ref, rs_ref):
    rs_ref[...] = jnp.sum(emb_ref[...], axis=1)[None, :]


def _vpass_body(wh_ref, wo_ref, v_ref):
    v_ref[...] = jnp.dot(wo_ref[...], wh_ref[...],
                         preferred_element_type=jnp.float32)


def _sc_gather_dot(idx_hbm, vw_hbm, table_hbm, out_hbm,
                   idx_v, vw_v, rows_v, acc_v, sem):
    wid = lax.axis_index("s") * 2 + lax.axis_index("c")
    pltpu.sync_copy(idx_hbm.at[wid], idx_v)
    pltpu.sync_copy(vw_hbm.at[wid], vw_v)
    copies = [
        pltpu.async_copy(table_hbm.at[idx_v.at[j]], rows_v.at[j], sem)
        for j in range(NCHUNK)
    ]
    for c in copies:
        c.wait()

    lane = lax.iota(jnp.int32, 16)
    acc = jnp.zeros((16,), jnp.float32)
    for j in range(NCHUNK):
        def body(g, a, j=j):
            pos = wid * PER_TILE + j * CHUNK + g * 16 + lane
            vals = rows_v[j, pl.ds(g * 16, 16)]
            w = vw_v[j, pl.ds(g * 16, 16)]
            w = jnp.where(pos < VOCAB, w, 0.0)
            return a + vals * w
        acc = lax.fori_loop(0, CHUNK // 16, body, acc)
    acc_v[...] = acc
    pltpu.sync_copy(acc_v, out_hbm.at[wid])


def _final_body(p_ref, vi_ref, img_ref, wo_ref, bh_ref, bo_ref, o_ref):
    word = jnp.sum(p_ref[...])
    img = jnp.sum(vi_ref[...] * img_ref[...])
    c = jnp.sum(wo_ref[...] * bh_ref[...]) + bo_ref[0, 0]
    x = word + img + c
    o_ref[...] = (1.0 / (1.0 + jnp.exp(-x))).reshape(1, 1)


def kernel(word_inputs, image_inputs, emb_table, W_h, b_h, W_o, b_o):
    rowsum2d = pl.pallas_call(
        _embsum_body,
        grid=(GRID1,),
        in_specs=[pl.BlockSpec((CH, HID), lambda i: (i, 0))],
        out_specs=pl.BlockSpec((1, CH), lambda i: (0, i)),
        out_shape=jax.ShapeDtypeStruct((1, PADDED), jnp.float32),
    )(emb_table)
    rowsum = rowsum2d.reshape(PADDED)

    v2d = pl.pallas_call(
        _vpass_body,
        grid=(GRID1,),
        in_specs=[
            pl.BlockSpec((HID, CH), lambda i: (0, i)),
            pl.BlockSpec((1, HID), lambda i: (0, 0)),
        ],
        out_specs=pl.BlockSpec((1, CH), lambda i: (0, i)),
        out_shape=jax.ShapeDtypeStruct((1, PADDED), jnp.float32),
    )(W_h, W_o)

    pad = PADDED - VOCAB
    idx_pad = jnp.concatenate(
        [word_inputs.astype(jnp.int32), jnp.zeros((pad,), jnp.int32)])
    idx3d = idx_pad.reshape(NTILES, NCHUNK, CHUNK)
    vw3d = v2d.reshape(NTILES, NCHUNK, CHUNK)
    v_img = lax.slice_in_dim(v2d.reshape(PADDED), VOCAB, TOTAL).reshape(1, IMG)

    mesh = plsc.VectorSubcoreMesh(core_axis_name="c", subcore_axis_name="s")
    partials = functools.partial(
        pl.kernel,
        mesh=mesh,
        out_type=jax.ShapeDtypeStruct((NTILES, 16), jnp.float32),
        scratch_types=[
            pltpu.VMEM((NCHUNK, CHUNK), jnp.int32),
            pltpu.VMEM((NCHUNK, CHUNK), jnp.float32),
            pltpu.VMEM((NCHUNK, CHUNK), jnp.float32),
            pltpu.VMEM((16,), jnp.float32),
            pltpu.SemaphoreType.DMA,
        ],
    )(_sc_gather_dot)(idx3d, vw3d, rowsum)

    out2d = pl.pallas_call(
        _final_body,
        out_shape=jax.ShapeDtypeStruct((1, 1), jnp.float32),
    )(partials, v_img, image_inputs.reshape(1, IMG), W_o,
      b_h.reshape(1, HID), b_o.reshape(1, 1))
    return out2d.reshape(1)


# speedup vs baseline: 1.1076x; 1.1076x over previous
import functools

import jax
import jax.numpy as jnp
from jax import lax
from jax.experimental import pallas as pl
from jax.experimental.pallas import tpu as pltpu
from jax.experimental.pallas import tpu_sc as plsc

VOCAB = 100000
IMG = 2048
HID = 128
TOTAL = VOCAB + IMG
PADDED = 102400

CH = 12800
GRID1 = 8

NTILES = 32
PER_TILE = PADDED // NTILES
CHUNK = 128
NCHUNK = PER_TILE // CHUNK


def _pass1_body(emb_ref, wh_ref, wo_ref, rs_ref, v_ref):
    rs_ref[...] = jnp.sum(emb_ref[...], axis=1)[None, :]
    v_ref[...] = jnp.dot(wo_ref[...], wh_ref[...],
                         preferred_element_type=jnp.float32)


def _sc_gather_dot(idx_hbm, vw_hbm, table_hbm, out_hbm,
                   idx_v, vw_v, rows_v, acc_v, sem):
    wid = lax.axis_index("s") * 2 + lax.axis_index("c")
    pltpu.sync_copy(idx_hbm.at[wid], idx_v)
    pltpu.sync_copy(vw_hbm.at[wid], vw_v)
    copies = [
        pltpu.async_copy(table_hbm.at[idx_v.at[j]], rows_v.at[j], sem)
        for j in range(NCHUNK)
    ]
    for c in copies:
        c.wait()

    lane = lax.iota(jnp.int32, 16)
    acc = jnp.zeros((16,), jnp.float32)
    for j in range(NCHUNK):
        def body(g, a, j=j):
            pos = wid * PER_TILE + j * CHUNK + g * 16 + lane
            vals = rows_v[j, pl.ds(g * 16, 16)]
            w = vw_v[j, pl.ds(g * 16, 16)]
            w = jnp.where(pos < VOCAB, w, 0.0)
            return a + vals * w
        acc = lax.fori_loop(0, CHUNK // 16, body, acc)
    acc_v[...] = acc
    pltpu.sync_copy(acc_v, out_hbm.at[wid])


def _final_body(p_ref, vi_ref, img_ref, wo_ref, bh_ref, bo_ref, o_ref):
    word = jnp.sum(p_ref[...])
    img = jnp.sum(vi_ref[...] * img_ref[...])
    c = jnp.sum(wo_ref[...] * bh_ref[...]) + bo_ref[0, 0]
    x = word + img + c
    o_ref[...] = (1.0 / (1.0 + jnp.exp(-x))).reshape(1, 1)


def kernel(word_inputs, image_inputs, emb_table, W_h, b_h, W_o, b_o):
    rowsum2d, v2d = pl.pallas_call(
        _pass1_body,
        grid=(GRID1,),
        in_specs=[
            pl.BlockSpec((CH, HID), lambda i: (i, 0)),
            pl.BlockSpec((HID, CH), lambda i: (0, i)),
            pl.BlockSpec((1, HID), lambda i: (0, 0)),
        ],
        out_specs=[
            pl.BlockSpec((1, CH), lambda i: (0, i)),
            pl.BlockSpec((1, CH), lambda i: (0, i)),
        ],
        out_shape=[
            jax.ShapeDtypeStruct((1, PADDED), jnp.float32),
            jax.ShapeDtypeStruct((1, PADDED), jnp.float32),
        ],
    )(emb_table, W_h, W_o)
    rowsum = rowsum2d.reshape(PADDED)

    pad = PADDED - VOCAB
    idx_pad = jnp.concatenate(
        [word_inputs.astype(jnp.int32), jnp.zeros((pad,), jnp.int32)])
    idx3d = idx_pad.reshape(NTILES, NCHUNK, CHUNK)
    vw3d = v2d.reshape(NTILES, NCHUNK, CHUNK)
    v_img = lax.slice_in_dim(v2d.reshape(PADDED), VOCAB, TOTAL).reshape(1, IMG)

    mesh = plsc.VectorSubcoreMesh(core_axis_name="c", subcore_axis_name="s")
    partials = functools.partial(
        pl.kernel,
        mesh=mesh,
        out_type=jax.ShapeDtypeStruct((NTILES, 16), jnp.float32),
        scratch_types=[
            pltpu.VMEM((NCHUNK, CHUNK), jnp.int32),
            pltpu.VMEM((NCHUNK, CHUNK), jnp.float32),
            pltpu.VMEM((NCHUNK, CHUNK), jnp.float32),
            pltpu.VMEM((16,), jnp.float32),
            pltpu.SemaphoreType.DMA,
        ],
    )(_sc_gather_dot)(idx3d, vw3d, rowsum)

    out2d = pl.pallas_call(
        _final_body,
        out_shape=jax.ShapeDtypeStruct((1, 1), jnp.float32),
    )(partials, v_img, image_inputs.reshape(1, IMG), W_o,
      b_h.reshape(1, HID), b_o.reshape(1, 1))
    return out2d.reshape(1)


# speedup vs baseline: 1.1169x; 1.0084x over previous
import functools

import jax
import jax.numpy as jnp
from jax import lax
from jax.experimental import pallas as pl
from jax.experimental.pallas import tpu as pltpu
from jax.experimental.pallas import tpu_sc as plsc

VOCAB = 100000
IMG = 2048
HID = 128
TOTAL = VOCAB + IMG
PADDED = 102400

CH = 12800
GRID1 = 8

NTILES = 32
PER_TILE = PADDED // NTILES
CHUNK = 128
NCHUNK = PER_TILE // CHUNK


def _pass1_body(emb_ref, wh_ref, wo_ref, rs_ref, v_ref):
    rs_ref[...] = jnp.sum(emb_ref[...], axis=1)[None, :]
    v_ref[...] = jnp.dot(wo_ref[...], wh_ref[...],
                         preferred_element_type=jnp.float32)


def _sc_gather_dot(idx_hbm, vw_hbm, table_hbm, out_hbm,
                   idx_v, vw_v, rows_v, acc_v, sem, sem_i, sem_w):
    wid = lax.axis_index("s") * 2 + lax.axis_index("c")
    idx_c = pltpu.async_copy(idx_hbm.at[wid], idx_v, sem_i)
    vw_c = pltpu.async_copy(vw_hbm.at[wid], vw_v, sem_w)
    idx_c.wait()
    copies = [
        pltpu.async_copy(table_hbm.at[idx_v.at[j]], rows_v.at[j], sem)
        for j in range(NCHUNK)
    ]
    vw_c.wait()

    lane = lax.iota(jnp.int32, 16)
    acc = jnp.zeros((16,), jnp.float32)
    for j in range(NCHUNK):
        copies[j].wait()

        def body(g, a, j=j):
            pos = wid * PER_TILE + j * CHUNK + g * 16 + lane
            vals = rows_v[j, pl.ds(g * 16, 16)]
            w = vw_v[j, pl.ds(g * 16, 16)]
            w = jnp.where(pos < VOCAB, w, 0.0)
            return a + vals * w
        acc = lax.fori_loop(0, CHUNK // 16, body, acc)
    acc_v[...] = acc
    pltpu.sync_copy(acc_v, out_hbm.at[wid])


def _final_body(p_ref, vi_ref, img_ref, wo_ref, bh_ref, bo_ref, o_ref):
    word = jnp.sum(p_ref[...])
    img = jnp.sum(vi_ref[...] * img_ref[...])
    c = jnp.sum(wo_ref[...] * bh_ref[...]) + bo_ref[0, 0]
    x = word + img + c
    o_ref[...] = (1.0 / (1.0 + jnp.exp(-x))).reshape(1, 1)


def kernel(word_inputs, image_inputs, emb_table, W_h, b_h, W_o, b_o):
    rowsum2d, v2d = pl.pallas_call(
        _pass1_body,
        grid=(GRID1,),
        in_specs=[
            pl.BlockSpec((CH, HID), lambda i: (i, 0)),
            pl.BlockSpec((HID, CH), lambda i: (0, i)),
            pl.BlockSpec((1, HID), lambda i: (0, 0)),
        ],
        out_specs=[
            pl.BlockSpec((1, CH), lambda i: (0, i)),
            pl.BlockSpec((1, CH), lambda i: (0, i)),
        ],
        out_shape=[
            jax.ShapeDtypeStruct((1, PADDED), jnp.float32),
            jax.ShapeDtypeStruct((1, PADDED), jnp.float32),
        ],
    )(emb_table, W_h, W_o)
    rowsum = rowsum2d.reshape(PADDED)

    pad = PADDED - VOCAB
    idx_pad = jnp.concatenate(
        [word_inputs.astype(jnp.int32), jnp.zeros((pad,), jnp.int32)])
    idx3d = idx_pad.reshape(NTILES, NCHUNK, CHUNK)
    vw3d = v2d.reshape(NTILES, NCHUNK, CHUNK)
    v_img = lax.slice_in_dim(v2d.reshape(PADDED), VOCAB, TOTAL).reshape(1, IMG)

    mesh = plsc.VectorSubcoreMesh(core_axis_name="c", subcore_axis_name="s")
    partials = functools.partial(
        pl.kernel,
        mesh=mesh,
        out_type=jax.ShapeDtypeStruct((NTILES, 16), jnp.float32),
        scratch_types=[
            pltpu.VMEM((NCHUNK, CHUNK), jnp.int32),
            pltpu.VMEM((NCHUNK, CHUNK), jnp.float32),
            pltpu.VMEM((NCHUNK, CHUNK), jnp.float32),
            pltpu.VMEM((16,), jnp.float32),
            pltpu.SemaphoreType.DMA,
            pltpu.SemaphoreType.DMA,
            pltpu.SemaphoreType.DMA,
        ],
    )(_sc_gather_dot)(idx3d, vw3d, rowsum)

    out2d = pl.pallas_call(
        _final_body,
        out_shape=jax.ShapeDtypeStruct((1, 1), jnp.float32),
    )(partials, v_img, image_inputs.reshape(1, IMG), W_o,
      b_h.reshape(1, HID), b_o.reshape(1, 1))
    return out2d.reshape(1)
